# baseline (speedup 1.0000x reference)
import functools
import math

import jax
import jax.numpy as jnp
from jax import lax
from jax.experimental import pallas as pl
from jax.experimental.pallas import tpu as pltpu
from jax.experimental.pallas import tpu_sc as plsc

N_NODES = 10000
EDGES = 320000
D = 128
DE = 16
G = 128
FD = 10

NW = 32
BB = 80
NBATCH = 128
EPW = BB * NBATCH
E_PAD = EPW * NW
N_PAD = 10240
RPT = N_PAD // 16
ZR = 8
NEG = -1.0e30

EB = 2048
RB = 1000
PB = 1000
BN_INV = 1.0 / math.sqrt(1.0 + 1e-5)


def _edge_mlp_body(ea_ref, w_ref, b_ref, o_ref):
    pid = pl.program_id(0)
    v = jnp.dot(ea_ref[...], w_ref[...], preferred_element_type=jnp.float32)
    v = v + b_ref[...]
    row = pid * EB + lax.broadcasted_iota(jnp.int32, (EB, 1), 0)
    o_ref[...] = jnp.where(row < EDGES, v, NEG)


_edge_mlp = pl.pallas_call(
    _edge_mlp_body,
    grid=(E_PAD // EB,),
    in_specs=[
        pl.BlockSpec((EB, DE), lambda i: (i, 0)),
        pl.BlockSpec((DE, D), lambda i: (0, 0)),
        pl.BlockSpec((1, D), lambda i: (0, 0)),
    ],
    out_specs=pl.BlockSpec((EB, D), lambda i: (i, 0)),
    out_shape=jax.ShapeDtypeStruct((E_PAD, D), jnp.float32),
)


def _msg_body(h_hbm, src_hbm, dst_hbm, e_hbm, out_hbm,
              sidx0, sidx1, grows0, erows0, grows1, erows1, didx0, didx1,
              agg, zbuf, sem0, sem1, semi0, semi1, semd0, semd1):
    c = lax.axis_index("c")
    s = lax.axis_index("s")
    wid = s * 2 + c
    row0 = s * RPT
    ebase = wid * EPW

    def issue_i(t, sidx, semi):
        pltpu.async_copy(src_hbm.at[pl.ds(ebase + t * BB, BB)], sidx, semi)

    def drain_i(t, sidx, semi):
        pltpu.make_async_copy(src_hbm.at[pl.ds(ebase + t * BB, BB)],
                              sidx, semi).wait()

    def issue_d(t, didx, semd):
        pltpu.async_copy(dst_hbm.at[pl.ds(ebase + t * BB, BB)], didx, semd)

    def drain_d(t, didx, semd):
        pltpu.make_async_copy(dst_hbm.at[pl.ds(ebase + t * BB, BB)],
                              didx, semd).wait()

    def issue(t, sidx, grows, erows, sem):
        pltpu.async_copy(h_hbm.at[sidx], grows, sem)
        pltpu.async_copy(e_hbm.at[pl.ds(ebase + t * BB, BB), :], erows, sem)

    def drain(t, sidx, grows, erows, sem):
        pltpu.make_async_copy(h_hbm.at[sidx], grows, sem).wait()
        pltpu.make_async_copy(e_hbm.at[pl.ds(ebase + t * BB, BB), :],
                              erows, sem).wait()

    def compute(grows, erows):
        def row_body(r, carry2):
            for j in range(D // 16):
                sl = pl.ds(j * 16, 16)
                v = grows[r, sl] + erows[r, sl]
                grows[r, sl] = jnp.maximum(v, 0.0)
            return carry2
        lax.fori_loop(0, BB, row_body, 0)

    issue_i(0, sidx0, semi0)
    issue_i(1, sidx1, semi1)
    issue_d(0, didx0, semd0)
    issue_d(1, didx1, semd1)
    drain_i(0, sidx0, semi0)
    issue(0, sidx0, grows0, erows0, sem0)

    zv = jnp.zeros((16,), jnp.float32)
    for r in range(ZR):
        for j in range(D // 16):
            zbuf[r, pl.ds(j * 16, 16)] = zv
    for k in range(RPT // ZR):
        pltpu.sync_copy(zbuf, agg.at[pl.ds(row0 + k * ZR, ZR), :])
    plsc.subcore_barrier()

    def batch_body(i, carry):
        t0 = 2 * i
        t1 = t0 + 1
        not_last = i < NBATCH // 2 - 1

        drain_i(t1, sidx1, semi1)
        issue(t1, sidx1, grows1, erows1, sem1)
        drain(t0, sidx0, grows0, erows0, sem0)

        @pl.when(not_last)
        def _():
            issue_i(t0 + 2, sidx0, semi0)

        compute(grows0, erows0)
        drain_d(t0, didx0, semd0)

        @pl.when(not_last)
        def _():
            issue_d(t0 + 2, didx0, semd0)
            drain_i(t0 + 2, sidx0, semi0)
            issue(t0 + 2, sidx0, grows0, erows0, sem0)

        drain(t1, sidx1, grows1, erows1, sem1)

        @pl.when(not_last)
        def _():
            issue_i(t1 + 2, sidx1, semi1)

        compute(grows1, erows1)
        drain_d(t1, didx1, semd1)

        @pl.when(not_last)
        def _():
            issue_d(t1 + 2, didx1, semd1)
        return carry

    lax.fori_loop(0, NBATCH // 2, batch_body, 0)
    plsc.subcore_barrier()
    pltpu.sync_copy(agg.at[pl.ds(row0, RPT), :],
                    out_hbm.at[c, pl.ds(row0, RPT), :])


_msg_kernel = functools.partial(
    pl.kernel,
    out_type=jax.ShapeDtypeStruct((2, N_PAD, D), jnp.float32),
    mesh=plsc.VectorSubcoreMesh(core_axis_name="c", subcore_axis_name="s"),
    scratch_types=[
        pltpu.VMEM((BB,), jnp.int32),
        pltpu.VMEM((BB,), jnp.int32),
        pltpu.VMEM((BB, D), jnp.float32),
        pltpu.VMEM((BB, D), jnp.float32),
        pltpu.VMEM((BB, D), jnp.float32),
        pltpu.VMEM((BB, D), jnp.float32),
        pltpu.VMEM((BB,), jnp.int32),
        pltpu.VMEM((BB,), jnp.int32),
        pltpu.VMEM_SHARED((N_PAD, D), jnp.float32),
        pltpu.VMEM((ZR, D), jnp.float32),
        pltpu.SemaphoreType.DMA,
        pltpu.SemaphoreType.DMA,
        pltpu.SemaphoreType.DMA,
        pltpu.SemaphoreType.DMA,
        pltpu.SemaphoreType.DMA,
        pltpu.SemaphoreType.DMA,
    ],
)(_msg_body)


def _node_mlp_body(h_ref, agg_ref, w1_ref, b1_ref, gs_ref, be_ref,
                   w2_ref, b2_ref, o_ref):
    z = h_ref[...] + agg_ref[0] + agg_ref[1]
    z = jnp.dot(z, w1_ref[...], preferred_element_type=jnp.float32)
    z = (z + b1_ref[...]) * (gs_ref[...] * BN_INV) + be_ref[...]
    z = jnp.maximum(z, 0.0)
    z = jnp.dot(z, w2_ref[...], preferred_element_type=jnp.float32)
    o_ref[...] = jnp.maximum(z + b2_ref[...], 0.0)


_node_mlp = pl.pallas_call(
    _node_mlp_body,
    grid=(N_NODES // RB,),
    in_specs=[
        pl.BlockSpec((RB, D), lambda i: (i, 0)),
        pl.BlockSpec((2, RB, D), lambda i: (0, i, 0)),
        pl.BlockSpec((D, D), lambda i: (0, 0)),
        pl.BlockSpec((1, D), lambda i: (0, 0)),
        pl.BlockSpec((1, D), lambda i: (0, 0)),
        pl.BlockSpec((1, D), lambda i: (0, 0)),
        pl.BlockSpec((D, D), lambda i: (0, 0)),
        pl.BlockSpec((1, D), lambda i: (0, 0)),
    ],
    out_specs=pl.BlockSpec((RB, D), lambda i: (i, 0)),
    out_shape=jax.ShapeDtypeStruct((N_NODES, D), jnp.float32),
)


def _pool_head_body(batch_ref, h1_ref, h2_ref, h3_ref, l1w_ref, l1b_ref,
                    l2w_ref, l2b_ref, o_ref, s1, s2, s3, cnt):
    pid = pl.program_id(0)
    oh = (lax.broadcasted_iota(jnp.int32, (G, PB), 0)
          == batch_ref[0]).astype(jnp.float32)

    @pl.when(pid == 0)
    def _():
        s1[...] = jnp.zeros_like(s1)
        s2[...] = jnp.zeros_like(s2)
        s3[...] = jnp.zeros_like(s3)
        cnt[...] = jnp.zeros_like(cnt)

    s1[...] += jnp.dot(oh, h1_ref[...], preferred_element_type=jnp.float32)
    s2[...] += jnp.dot(oh, h2_ref[...], preferred_element_type=jnp.float32)
    s3[...] += jnp.dot(oh, h3_ref[...], preferred_element_type=jnp.float32)
    cnt[...] += jnp.sum(oh, axis=1, keepdims=True)

    @pl.when(pid == (N_NODES // PB) - 1)
    def _():
        c = jnp.maximum(cnt[...], 1.0)
        hh = jnp.concatenate([s1[...] / c, s2[...] / c, s3[...] / c], axis=1)
        hh = jnp.dot(hh, l1w_ref[...], preferred_element_type=jnp.float32)
        hh = jnp.maximum(hh + l1b_ref[...], 0.0)
        hh = jnp.dot(hh, l2w_ref[...], preferred_element_type=jnp.float32)
        hh = hh + l2b_ref[...]
        m = jnp.max(hh, axis=1, keepdims=True)
        lse = m + jnp.log(jnp.sum(jnp.exp(hh - m), axis=1, keepdims=True))
        o_ref[...] = hh - lse


_pool_head = pl.pallas_call(
    _pool_head_body,
    grid=(N_NODES // PB,),
    in_specs=[
        pl.BlockSpec((1, 1, PB), lambda i: (i, 0, 0)),
        pl.BlockSpec((PB, D), lambda i: (i, 0)),
        pl.BlockSpec((PB, D), lambda i: (i, 0)),
        pl.BlockSpec((PB, D), lambda i: (i, 0)),
        pl.BlockSpec((3 * D, 3 * D), lambda i: (0, 0)),
        pl.BlockSpec((1, 3 * D), lambda i: (0, 0)),
        pl.BlockSpec((3 * D, FD), lambda i: (0, 0)),
        pl.BlockSpec((1, FD), lambda i: (0, 0)),
    ],
    out_specs=pl.BlockSpec((G, FD), lambda i: (0, 0)),
    out_shape=jax.ShapeDtypeStruct((G, FD), jnp.float32),
    scratch_shapes=[
        pltpu.VMEM((G, D), jnp.float32),
        pltpu.VMEM((G, D), jnp.float32),
        pltpu.VMEM((G, D), jnp.float32),
        pltpu.VMEM((G, 1), jnp.float32),
    ],
)


def kernel(x, edge_index, edge_attr, batch, params):
    p = params
    ei = jnp.pad(edge_index, ((0, 0), (0, E_PAD - EDGES)))
    src = ei[0]
    dst = ei[1]
    ea = jnp.pad(edge_attr, ((0, E_PAD - EDGES), (0, 0)))
    batch3 = batch.reshape(N_NODES // PB, 1, PB)

    h = x
    hs = []
    for pre in ("c1", "c2", "c3"):
        e = _edge_mlp(ea, p[pre + 'ew'], p[pre + 'eb'].reshape(1, D))
        agg = _msg_kernel(h, src, dst, e)
        h = _node_mlp(h, agg, p[pre + 'w1'], p[pre + 'b1'].reshape(1, D),
                      p[pre + 'g'].reshape(1, D), p[pre + 'be'].reshape(1, D),
                      p[pre + 'w2'], p[pre + 'b2'].reshape(1, D))
        hs.append(h)

    return _pool_head(batch3, hs[0], hs[1], hs[2], p['l1w'],
                      p['l1b'].reshape(1, 3 * D), p['l2w'],
                      p['l2b'].reshape(1, FD))


# speedup vs baseline: 1.1622x; 1.1622x over previous
import functools
import math

import jax
import jax.numpy as jnp
from jax import lax
from jax.experimental import pallas as pl
from jax.experimental.pallas import tpu as pltpu
from jax.experimental.pallas import tpu_sc as plsc

N_NODES = 10000
EDGES = 320000
D = 128
DE = 16
G = 128
FD = 10

NW = 32
BB = 80
NBT = 256
NB0 = 182
NB1 = NBT - NB0
E_PAD = NBT * BB * 16
N_PAD = 10240
RPT = N_PAD // 16
ZR = 8
NEG = -1.0e30

EB = 2048
RB = 1000
PB = 1000
BN_INV = 1.0 / math.sqrt(1.0 + 1e-5)


def _edge_mlp_body(ea_ref, w_ref, b_ref, o_ref):
    pid = pl.program_id(0)
    v = jnp.dot(ea_ref[...], w_ref[...], preferred_element_type=jnp.float32)
    v = v + b_ref[...]
    row = pid * EB + lax.broadcasted_iota(jnp.int32, (EB, 1), 0)
    o_ref[...] = jnp.where(row < EDGES, v, NEG)


_edge_mlp = pl.pallas_call(
    _edge_mlp_body,
    grid=(E_PAD // EB,),
    in_specs=[
        pl.BlockSpec((EB, DE), lambda i: (i, 0)),
        pl.BlockSpec((DE, D), lambda i: (0, 0)),
        pl.BlockSpec((1, D), lambda i: (0, 0)),
    ],
    out_specs=pl.BlockSpec((EB, D), lambda i: (i, 0)),
    out_shape=jax.ShapeDtypeStruct((E_PAD, D), jnp.float32),
)


def _msg_body(h_hbm, src_hbm, dst_hbm, e_hbm, out_hbm,
              sidx0, sidx1, grows0, erows0, grows1, erows1, didx0, didx1,
              agg, zbuf, sem0, sem1, semi0, semi1, semd0, semd1):
    c = lax.axis_index("c")
    s = lax.axis_index("s")
    row0 = s * RPT
    ebase = (s * NBT + c * NB0) * BB
    nhalf = jnp.where(c == 0, NB0 // 2, NB1 // 2)

    def issue_i(t, sidx, semi):
        pltpu.async_copy(src_hbm.at[pl.ds(ebase + t * BB, BB)], sidx, semi)

    def drain_i(t, sidx, semi):
        pltpu.make_async_copy(src_hbm.at[pl.ds(ebase + t * BB, BB)],
                              sidx, semi).wait()

    def issue_d(t, didx, semd):
        pltpu.async_copy(dst_hbm.at[pl.ds(ebase + t * BB, BB)], didx, semd)

    def drain_d(t, didx, semd):
        pltpu.make_async_copy(dst_hbm.at[pl.ds(ebase + t * BB, BB)],
                              didx, semd).wait()

    def issue(t, sidx, grows, erows, sem):
        pltpu.async_copy(h_hbm.at[sidx], grows, sem)
        pltpu.async_copy(e_hbm.at[pl.ds(ebase + t * BB, BB), :], erows, sem)

    def drain(t, sidx, grows, erows, sem):
        pltpu.make_async_copy(h_hbm.at[sidx], grows, sem).wait()
        pltpu.make_async_copy(e_hbm.at[pl.ds(ebase + t * BB, BB), :],
                              erows, sem).wait()

    def compute(grows, erows):
        def row_body(r, carry2):
            for j in range(D // 16):
                sl = pl.ds(j * 16, 16)
                v = grows[r, sl] + erows[r, sl]
                grows[r, sl] = jnp.maximum(v, 0.0)
            return carry2
        lax.fori_loop(0, BB, row_body, 0)

    issue_i(0, sidx0, semi0)
    issue_i(1, sidx1, semi1)
    issue_d(0, didx0, semd0)
    issue_d(1, didx1, semd1)
    drain_i(0, sidx0, semi0)
    issue(0, sidx0, grows0, erows0, sem0)

    zv = jnp.zeros((16,), jnp.float32)
    for r in range(ZR):
        for j in range(D // 16):
            zbuf[r, pl.ds(j * 16, 16)] = zv
    for k in range(RPT // ZR):
        pltpu.sync_copy(zbuf, agg.at[pl.ds(row0 + k * ZR, ZR), :])
    plsc.subcore_barrier()

    def batch_body(i, carry):
        t0 = 2 * i
        t1 = t0 + 1
        not_last = i < nhalf - 1

        drain_i(t1, sidx1, semi1)
        issue(t1, sidx1, grows1, erows1, sem1)
        drain(t0, sidx0, grows0, erows0, sem0)

        @pl.when(not_last)
        def _():
            issue_i(t0 + 2, sidx0, semi0)

        compute(grows0, erows0)
        drain_d(t0, didx0, semd0)
        pltpu.sync_copy(grows0, agg.at[didx0], add=True)

        @pl.when(not_last)
        def _():
            issue_d(t0 + 2, didx0, semd0)
            drain_i(t0 + 2, sidx0, semi0)
            issue(t0 + 2, sidx0, grows0, erows0, sem0)

        drain(t1, sidx1, grows1, erows1, sem1)

        @pl.when(not_last)
        def _():
            issue_i(t1 + 2, sidx1, semi1)

        compute(grows1, erows1)
        drain_d(t1, didx1, semd1)
        pltpu.sync_copy(grows1, agg.at[didx1], add=True)

        @pl.when(not_last)
        def _():
            issue_d(t1 + 2, didx1, semd1)
        return carry

    lax.fori_loop(0, nhalf, batch_body, 0)
    plsc.subcore_barrier()
    pltpu.sync_copy(agg.at[pl.ds(row0, RPT), :],
                    out_hbm.at[c, pl.ds(row0, RPT), :])


_msg_kernel = functools.partial(
    pl.kernel,
    out_type=jax.ShapeDtypeStruct((2, N_PAD, D), jnp.float32),
    mesh=plsc.VectorSubcoreMesh(core_axis_name="c", subcore_axis_name="s"),
    scratch_types=[
        pltpu.VMEM((BB,), jnp.int32),
        pltpu.VMEM((BB,), jnp.int32),
        pltpu.VMEM((BB, D), jnp.float32),
        pltpu.VMEM((BB, D), jnp.float32),
        pltpu.VMEM((BB, D), jnp.float32),
        pltpu.VMEM((BB, D), jnp.float32),
        pltpu.VMEM((BB,), jnp.int32),
        pltpu.VMEM((BB,), jnp.int32),
        pltpu.VMEM_SHARED((N_PAD, D), jnp.float32),
        pltpu.VMEM((ZR, D), jnp.float32),
        pltpu.SemaphoreType.DMA,
        pltpu.SemaphoreType.DMA,
        pltpu.SemaphoreType.DMA,
        pltpu.SemaphoreType.DMA,
        pltpu.SemaphoreType.DMA,
        pltpu.SemaphoreType.DMA,
    ],
)(_msg_body)


def _node_mlp_body(h_ref, agg_ref, w1_ref, b1_ref, gs_ref, be_ref,
                   w2_ref, b2_ref, o_ref):
    z = h_ref[...] + agg_ref[0] + agg_ref[1]
    z = jnp.dot(z, w1_ref[...], preferred_element_type=jnp.float32)
    z = (z + b1_ref[...]) * (gs_ref[...] * BN_INV) + be_ref[...]
    z = jnp.maximum(z, 0.0)
    z = jnp.dot(z, w2_ref[...], preferred_element_type=jnp.float32)
    o_ref[...] = jnp.maximum(z + b2_ref[...], 0.0)


_node_mlp = pl.pallas_call(
    _node_mlp_body,
    grid=(N_NODES // RB,),
    in_specs=[
        pl.BlockSpec((RB, D), lambda i: (i, 0)),
        pl.BlockSpec((2, RB, D), lambda i: (0, i, 0)),
        pl.BlockSpec((D, D), lambda i: (0, 0)),
        pl.BlockSpec((1, D), lambda i: (0, 0)),
        pl.BlockSpec((1, D), lambda i: (0, 0)),
        pl.BlockSpec((1, D), lambda i: (0, 0)),
        pl.BlockSpec((D, D), lambda i: (0, 0)),
        pl.BlockSpec((1, D), lambda i: (0, 0)),
    ],
    out_specs=pl.BlockSpec((RB, D), lambda i: (i, 0)),
    out_shape=jax.ShapeDtypeStruct((N_NODES, D), jnp.float32),
)


def _pool_head_body(batch_ref, h1_ref, h2_ref, h3_ref, l1w_ref, l1b_ref,
                    l2w_ref, l2b_ref, o_ref, s1, s2, s3, cnt):
    pid = pl.program_id(0)
    oh = (lax.broadcasted_iota(jnp.int32, (G, PB), 0)
          == batch_ref[0]).astype(jnp.float32)

    @pl.when(pid == 0)
    def _():
        s1[...] = jnp.zeros_like(s1)
        s2[...] = jnp.zeros_like(s2)
        s3[...] = jnp.zeros_like(s3)
        cnt[...] = jnp.zeros_like(cnt)

    s1[...] += jnp.dot(oh, h1_ref[...], preferred_element_type=jnp.float32)
    s2[...] += jnp.dot(oh, h2_ref[...], preferred_element_type=jnp.float32)
    s3[...] += jnp.dot(oh, h3_ref[...], preferred_element_type=jnp.float32)
    cnt[...] += jnp.sum(oh, axis=1, keepdims=True)

    @pl.when(pid == (N_NODES // PB) - 1)
    def _():
        c = jnp.maximum(cnt[...], 1.0)
        hh = jnp.concatenate([s1[...] / c, s2[...] / c, s3[...] / c], axis=1)
        hh = jnp.dot(hh, l1w_ref[...], preferred_element_type=jnp.float32)
        hh = jnp.maximum(hh + l1b_ref[...], 0.0)
        hh = jnp.dot(hh, l2w_ref[...], preferred_element_type=jnp.float32)
        hh = hh + l2b_ref[...]
        m = jnp.max(hh, axis=1, keepdims=True)
        lse = m + jnp.log(jnp.sum(jnp.exp(hh - m), axis=1, keepdims=True))
        o_ref[...] = hh - lse


_pool_head = pl.pallas_call(
    _pool_head_body,
    grid=(N_NODES // PB,),
    in_specs=[
        pl.BlockSpec((1, 1, PB), lambda i: (i, 0, 0)),
        pl.BlockSpec((PB, D), lambda i: (i, 0)),
        pl.BlockSpec((PB, D), lambda i: (i, 0)),
        pl.BlockSpec((PB, D), lambda i: (i, 0)),
        pl.BlockSpec((3 * D, 3 * D), lambda i: (0, 0)),
        pl.BlockSpec((1, 3 * D), lambda i: (0, 0)),
        pl.BlockSpec((3 * D, FD), lambda i: (0, 0)),
        pl.BlockSpec((1, FD), lambda i: (0, 0)),
    ],
    out_specs=pl.BlockSpec((G, FD), lambda i: (0, 0)),
    out_shape=jax.ShapeDtypeStruct((G, FD), jnp.float32),
    scratch_shapes=[
        pltpu.VMEM((G, D), jnp.float32),
        pltpu.VMEM((G, D), jnp.float32),
        pltpu.VMEM((G, D), jnp.float32),
        pltpu.VMEM((G, 1), jnp.float32),
    ],
)


def kernel(x, edge_index, edge_attr, batch, params):
    p = params
    ei = jnp.pad(edge_index, ((0, 0), (0, E_PAD - EDGES)))
    src = ei[0]
    dst = ei[1]
    ea = jnp.pad(edge_attr, ((0, E_PAD - EDGES), (0, 0)))
    batch3 = batch.reshape(N_NODES // PB, 1, PB)

    h = x
    hs = []
    for pre in ("c1", "c2", "c3"):
        e = _edge_mlp(ea, p[pre + 'ew'], p[pre + 'eb'].reshape(1, D))
        agg = _msg_kernel(h, src, dst, e)
        h = _node_mlp(h, agg, p[pre + 'w1'], p[pre + 'b1'].reshape(1, D),
                      p[pre + 'g'].reshape(1, D), p[pre + 'be'].reshape(1, D),
                      p[pre + 'w2'], p[pre + 'b2'].reshape(1, D))
        hs.append(h)

    return _pool_head(batch3, hs[0], hs[1], hs[2], p['l1w'],
                      p['l1b'].reshape(1, 3 * D), p['l2w'],
                      p['l2b'].reshape(1, FD))


# speedup vs baseline: 1.8642x; 1.6041x over previous
import functools
import math

import jax
import jax.numpy as jnp
from jax import lax
from jax.experimental import pallas as pl
from jax.experimental.pallas import tpu as pltpu
from jax.experimental.pallas import tpu_sc as plsc

N_NODES = 10000
EDGES = 320000
D = 128
DE = 16
G = 128
FD = 10

NW = 32
BB = 80
NBT = 250
NB0 = 178
NB1 = NBT - NB0
N_PAD = 10240
RPT = N_PAD // 16
ZR = 8
NEG = -1.0e30

EB = 4000
RB = 1000
PB = 1000
BN_INV = 1.0 / math.sqrt(1.0 + 1e-5)


def _edge_mlp_body(ea_ref, w_ref, b_ref, o_ref):
    v = jnp.dot(ea_ref[...], w_ref[...], preferred_element_type=jnp.float32)
    o_ref[...] = v + b_ref[...]


_edge_mlp = pl.pallas_call(
    _edge_mlp_body,
    grid=(EDGES // EB,),
    in_specs=[
        pl.BlockSpec((EB, DE), lambda i: (i, 0)),
        pl.BlockSpec((DE, D), lambda i: (0, 0)),
        pl.BlockSpec((1, D), lambda i: (0, 0)),
    ],
    out_specs=pl.BlockSpec((EB, D), lambda i: (i, 0)),
    out_shape=jax.ShapeDtypeStruct((EDGES, D), jnp.float32),
)


def _msg_body(h_hbm, src_hbm, dst_hbm, e_hbm, out_hbm,
              sidx0, sidx1, grows0, erows0, grows1, erows1, didx0, didx1,
              agg, zbuf, sem0, sem1, semi0, semi1, semd0, semd1):
    c = lax.axis_index("c")
    s = lax.axis_index("s")
    row0 = s * RPT
    ebase = (s * NBT + c * NB0) * BB
    nhalf = jnp.where(c == 0, NB0 // 2, NB1 // 2)

    def issue_i(t, sidx, semi):
        pltpu.async_copy(src_hbm.at[pl.ds(ebase + t * BB, BB)], sidx, semi)

    def drain_i(t, sidx, semi):
        pltpu.make_async_copy(src_hbm.at[pl.ds(ebase + t * BB, BB)],
                              sidx, semi).wait()

    def issue_d(t, didx, semd):
        pltpu.async_copy(dst_hbm.at[pl.ds(ebase + t * BB, BB)], didx, semd)

    def drain_d(t, didx, semd):
        pltpu.make_async_copy(dst_hbm.at[pl.ds(ebase + t * BB, BB)],
                              didx, semd).wait()

    def issue(t, sidx, grows, erows, sem):
        pltpu.async_copy(h_hbm.at[sidx], grows, sem)
        pltpu.async_copy(e_hbm.at[pl.ds(ebase + t * BB, BB), :], erows, sem)

    def drain(t, sidx, grows, erows, sem):
        pltpu.make_async_copy(h_hbm.at[sidx], grows, sem).wait()
        pltpu.make_async_copy(e_hbm.at[pl.ds(ebase + t * BB, BB), :],
                              erows, sem).wait()

    def compute(grows, erows):
        @plsc.parallel_loop(0, BB, step=1, unroll=2)
        def row_body(r):
            for j in range(D // 16):
                sl = pl.ds(j * 16, 16)
                v = grows[r, sl] + erows[r, sl]
                grows[r, sl] = jnp.maximum(v, 0.0)

    issue_i(0, sidx0, semi0)
    issue_i(1, sidx1, semi1)
    issue_d(0, didx0, semd0)
    issue_d(1, didx1, semd1)
    drain_i(0, sidx0, semi0)
    issue(0, sidx0, grows0, erows0, sem0)

    zv = jnp.zeros((16,), jnp.float32)
    for r in range(ZR):
        for j in range(D // 16):
            zbuf[r, pl.ds(j * 16, 16)] = zv
    for k in range(RPT // ZR):
        pltpu.sync_copy(zbuf, agg.at[pl.ds(row0 + k * ZR, ZR), :])
    plsc.subcore_barrier()

    def batch_body(i, carry):
        t0 = 2 * i
        t1 = t0 + 1
        not_last = i < nhalf - 1

        drain_i(t1, sidx1, semi1)
        issue(t1, sidx1, grows1, erows1, sem1)
        drain(t0, sidx0, grows0, erows0, sem0)

        @pl.when(not_last)
        def _():
            issue_i(t0 + 2, sidx0, semi0)

        compute(grows0, erows0)
        drain_d(t0, didx0, semd0)
        pltpu.sync_copy(grows0, agg.at[didx0], add=True)

        @pl.when(not_last)
        def _():
            issue_d(t0 + 2, didx0, semd0)
            drain_i(t0 + 2, sidx0, semi0)
            issue(t0 + 2, sidx0, grows0, erows0, sem0)

        drain(t1, sidx1, grows1, erows1, sem1)

        @pl.when(not_last)
        def _():
            issue_i(t1 + 2, sidx1, semi1)

        compute(grows1, erows1)
        drain_d(t1, didx1, semd1)
        pltpu.sync_copy(grows1, agg.at[didx1], add=True)

        @pl.when(not_last)
        def _():
            issue_d(t1 + 2, didx1, semd1)
        return carry

    lax.fori_loop(0, nhalf, batch_body, 0)
    plsc.subcore_barrier()
    pltpu.sync_copy(agg.at[pl.ds(row0, RPT), :],
                    out_hbm.at[c, pl.ds(row0, RPT), :])


_msg_kernel = functools.partial(
    pl.kernel,
    out_type=jax.ShapeDtypeStruct((2, N_PAD, D), jnp.float32),
    mesh=plsc.VectorSubcoreMesh(core_axis_name="c", subcore_axis_name="s"),
    scratch_types=[
        pltpu.VMEM((BB,), jnp.int32),
        pltpu.VMEM((BB,), jnp.int32),
        pltpu.VMEM((BB, D), jnp.float32),
        pltpu.VMEM((BB, D), jnp.float32),
        pltpu.VMEM((BB, D), jnp.float32),
        pltpu.VMEM((BB, D), jnp.float32),
        pltpu.VMEM((BB,), jnp.int32),
        pltpu.VMEM((BB,), jnp.int32),
        pltpu.VMEM_SHARED((N_PAD, D), jnp.float32),
        pltpu.VMEM((ZR, D), jnp.float32),
        pltpu.SemaphoreType.DMA,
        pltpu.SemaphoreType.DMA,
        pltpu.SemaphoreType.DMA,
        pltpu.SemaphoreType.DMA,
        pltpu.SemaphoreType.DMA,
        pltpu.SemaphoreType.DMA,
    ],
)(_msg_body)


def _node_mlp_body(h_ref, agg_ref, w1_ref, b1_ref, gs_ref, be_ref,
                   w2_ref, b2_ref, o_ref):
    z = h_ref[...] + agg_ref[0] + agg_ref[1]
    z = jnp.dot(z, w1_ref[...], preferred_element_type=jnp.float32)
    z = (z + b1_ref[...]) * (gs_ref[...] * BN_INV) + be_ref[...]
    z = jnp.maximum(z, 0.0)
    z = jnp.dot(z, w2_ref[...], preferred_element_type=jnp.float32)
    o_ref[...] = jnp.maximum(z + b2_ref[...], 0.0)


_node_mlp = pl.pallas_call(
    _node_mlp_body,
    grid=(N_NODES // RB,),
    in_specs=[
        pl.BlockSpec((RB, D), lambda i: (i, 0)),
        pl.BlockSpec((2, RB, D), lambda i: (0, i, 0)),
        pl.BlockSpec((D, D), lambda i: (0, 0)),
        pl.BlockSpec((1, D), lambda i: (0, 0)),
        pl.BlockSpec((1, D), lambda i: (0, 0)),
        pl.BlockSpec((1, D), lambda i: (0, 0)),
        pl.BlockSpec((D, D), lambda i: (0, 0)),
        pl.BlockSpec((1, D), lambda i: (0, 0)),
    ],
    out_specs=pl.BlockSpec((RB, D), lambda i: (i, 0)),
    out_shape=jax.ShapeDtypeStruct((N_NODES, D), jnp.float32),
)


def _pool_head_body(batch_ref, h1_ref, h2_ref, h3_ref, l1w_ref, l1b_ref,
                    l2w_ref, l2b_ref, o_ref, s1, s2, s3, cnt):
    pid = pl.program_id(0)
    oh = (lax.broadcasted_iota(jnp.int32, (G, PB), 0)
          == batch_ref[0]).astype(jnp.float32)

    @pl.when(pid == 0)
    def _():
        s1[...] = jnp.zeros_like(s1)
        s2[...] = jnp.zeros_like(s2)
        s3[...] = jnp.zeros_like(s3)
        cnt[...] = jnp.zeros_like(cnt)

    s1[...] += jnp.dot(oh, h1_ref[...], preferred_element_type=jnp.float32)
    s2[...] += jnp.dot(oh, h2_ref[...], preferred_element_type=jnp.float32)
    s3[...] += jnp.dot(oh, h3_ref[...], preferred_element_type=jnp.float32)
    cnt[...] += jnp.sum(oh, axis=1, keepdims=True)

    @pl.when(pid == (N_NODES // PB) - 1)
    def _():
        c = jnp.maximum(cnt[...], 1.0)
        hh = jnp.concatenate([s1[...] / c, s2[...] / c, s3[...] / c], axis=1)
        hh = jnp.dot(hh, l1w_ref[...], preferred_element_type=jnp.float32)
        hh = jnp.maximum(hh + l1b_ref[...], 0.0)
        hh = jnp.dot(hh, l2w_ref[...], preferred_element_type=jnp.float32)
        hh = hh + l2b_ref[...]
        m = jnp.max(hh, axis=1, keepdims=True)
        lse = m + jnp.log(jnp.sum(jnp.exp(hh - m), axis=1, keepdims=True))
        o_ref[...] = hh - lse


_pool_head = pl.pallas_call(
    _pool_head_body,
    grid=(N_NODES // PB,),
    in_specs=[
        pl.BlockSpec((1, 1, PB), lambda i: (i, 0, 0)),
        pl.BlockSpec((PB, D), lambda i: (i, 0)),
        pl.BlockSpec((PB, D), lambda i: (i, 0)),
        pl.BlockSpec((PB, D), lambda i: (i, 0)),
        pl.BlockSpec((3 * D, 3 * D), lambda i: (0, 0)),
        pl.BlockSpec((1, 3 * D), lambda i: (0, 0)),
        pl.BlockSpec((3 * D, FD), lambda i: (0, 0)),
        pl.BlockSpec((1, FD), lambda i: (0, 0)),
    ],
    out_specs=pl.BlockSpec((G, FD), lambda i: (0, 0)),
    out_shape=jax.ShapeDtypeStruct((G, FD), jnp.float32),
    scratch_shapes=[
        pltpu.VMEM((G, D), jnp.float32),
        pltpu.VMEM((G, D), jnp.float32),
        pltpu.VMEM((G, D), jnp.float32),
        pltpu.VMEM((G, 1), jnp.float32),
    ],
)


def kernel(x, edge_index, edge_attr, batch, params):
    p = params
    src = edge_index[0]
    dst = edge_index[1]
    batch3 = batch.reshape(N_NODES // PB, 1, PB)

    h = x
    hs = []
    for pre in ("c1", "c2", "c3"):
        e = _edge_mlp(edge_attr, p[pre + 'ew'], p[pre + 'eb'].reshape(1, D))
        agg = _msg_kernel(h, src, dst, e)
        h = _node_mlp(h, agg, p[pre + 'w1'], p[pre + 'b1'].reshape(1, D),
                      p[pre + 'g'].reshape(1, D), p[pre + 'be'].reshape(1, D),
                      p[pre + 'w2'], p[pre + 'b2'].reshape(1, D))
        hs.append(h)

    return _pool_head(batch3, hs[0], hs[1], hs[2], p['l1w'],
                      p['l1b'].reshape(1, 3 * D), p['l2w'],
                      p['l2b'].reshape(1, FD))
